# baseline (speedup 1.0000x reference)
import functools

import jax
import jax.numpy as jnp
from jax import lax
from jax.experimental import pallas as pl
from jax.experimental.pallas import tpu as pltpu
from jax.experimental.pallas import tpu_sc as plsc

H = 100
W = 200
N = H * W
NUM_CLASSES = 80
MAX_OUT = 300
NEG_INF = float("-inf")
BIG_F = 1.0e9

GR = 112
GP = 256
NPIX = GR * GP
NT = 16
RPT = GR // NT
STRIPE = RPT * GP
ROW_VREGS = 13
WRAD = 7
OUT_ROWS = 304


def _prep_body(cls_ref, ctr_ref, reg_ref, cen_ref, thr_ref,
               s_ref, x1_ref, y1_ref, x2_ref, y2_ref, area_ref, ids_ref):
    thr = thr_ref[0, 0]

    def cls_step(c, carry):
        acc, amax = carry
        x = cls_ref[c]
        gt = x > acc
        acc = jnp.where(gt, x, acc)
        amax = jnp.where(gt, c, amax)
        return acc, amax

    acc0 = cls_ref[0]
    amax0 = jnp.zeros((GR, GP), jnp.int32)
    cls_scores, cls_ids = jax.lax.fori_loop(1, NUM_CLASSES, cls_step, (acc0, amax0))

    score = cls_scores * ctr_ref[...]
    gi = jax.lax.broadcasted_iota(jnp.int32, (GR, GP), 0)
    ci = jax.lax.broadcasted_iota(jnp.int32, (GR, GP), 1)
    valid = (gi < H) & (ci < W)
    s_ref[...] = jnp.where((score > thr) & valid, score, NEG_INF)

    cx = cen_ref[0]
    cy = cen_ref[1]
    x1 = cx - reg_ref[0]
    y1 = cy - reg_ref[1]
    x2 = cx + reg_ref[2]
    y2 = cy + reg_ref[3]
    x1_ref[...] = x1
    y1_ref[...] = y1
    x2_ref[...] = x2
    y2_ref[...] = y2
    area_ref[...] = (x2 - x1) * (y2 - y1)
    ids_ref[...] = cls_ids.astype(jnp.float32)


def _nms_sc_body(s_hbm, x1_hbm, y1_hbm, x2_hbm, y2_hbm, area_hbm, ids_hbm,
                 par_hbm, out_hbm,
                 s_v, x1_v, y1_v, x2_v, y2_v, area_v, ids_v,
                 par_v, rec_v, all_rec_v, wrec_v, out_v,
                 rmax_v, ridx_v, shared):
    cid = lax.axis_index("c")
    sid = lax.axis_index("s")
    base = sid * STRIPE
    base_row = sid * RPT

    pltpu.sync_copy(s_hbm.at[pl.ds(base, STRIPE)], s_v)
    pltpu.sync_copy(x1_hbm.at[pl.ds(base, STRIPE)], x1_v)
    pltpu.sync_copy(y1_hbm.at[pl.ds(base, STRIPE)], y1_v)
    pltpu.sync_copy(x2_hbm.at[pl.ds(base, STRIPE)], x2_v)
    pltpu.sync_copy(y2_hbm.at[pl.ds(base, STRIPE)], y2_v)
    pltpu.sync_copy(area_hbm.at[pl.ds(base, STRIPE)], area_v)
    pltpu.sync_copy(ids_hbm.at[pl.ds(base, STRIPE)], ids_v)
    pltpu.sync_copy(par_hbm, par_v)

    li = lax.iota(jnp.int32, 16)
    lif = li.astype(jnp.float32)
    iou_thr = par_v[...]
    zero16 = jnp.zeros((16,), jnp.int32)
    neginf16 = jnp.full((16,), NEG_INF, jnp.float32)

    def scan_row(r_local):
        g_orig = (base_row + r_local) * W
        rb = r_local * GP
        acc = s_v[pl.ds(rb, 16)]
        idxv = jnp.float32(1.0) * g_orig + lif
        for t in range(1, ROW_VREGS):
            sv = s_v[pl.ds(rb + t * 16, 16)]
            gt = sv > acc
            acc = jnp.where(gt, sv, acc)
            idxv = jnp.where(gt, jnp.float32(1.0) * (g_orig + t * 16) + lif, idxv)
        rm = jnp.max(acc)
        ri = jnp.min(jnp.where(acc == rm, idxv, BIG_F))
        return rm, ri

    rmax = neginf16
    ridx = jnp.full((16,), BIG_F, jnp.float32)
    for r in range(RPT):
        rm, ri = scan_row(r)
        rmax = jnp.where(li == r, rm, rmax)
        ridx = jnp.where(li == r, ri, ridx)
    rmax_v[...] = rmax
    ridx_v[...] = ridx

    def step(k, carry):
        rmax = rmax_v[...]
        ridx = ridx_v[...]
        m = jnp.max(rmax)
        idxf = jnp.min(jnp.where(rmax == m, ridx, BIG_F))
        lidx = idxf.astype(jnp.int32)
        lg = lidx // W - base_row
        lpos = lg * GP + lidx % W
        lpos16 = zero16 + lpos
        gx1 = plsc.load_gather(x1_v, [lpos16])
        gy1 = plsc.load_gather(y1_v, [lpos16])
        gx2 = plsc.load_gather(x2_v, [lpos16])
        gy2 = plsc.load_gather(y2_v, [lpos16])
        gar = plsc.load_gather(area_v, [lpos16])
        gid = plsc.load_gather(ids_v, [lpos16])
        rec = jnp.where(li == 0, m,
              jnp.where(li == 1, idxf,
              jnp.where(li == 2, gx1,
              jnp.where(li == 3, gy1,
              jnp.where(li == 4, gx2,
              jnp.where(li == 5, gy2,
              jnp.where(li == 6, gid,
              jnp.where(li == 7, gar, 0.0))))))))
        rec_v[...] = rec

        par = lax.rem(k, 2)
        slot = par * (NT * 16) + sid * 16
        pltpu.sync_copy(rec_v, shared.at[pl.ds(slot, 16)])
        plsc.subcore_barrier()
        pltpu.sync_copy(shared.at[pl.ds(par * (NT * 16), NT * 16)], all_rec_v)

        scores = plsc.load_gather(all_rec_v, [li * 16 + 0])
        gidxs = plsc.load_gather(all_rec_v, [li * 16 + 1])
        wm = jnp.max(scores)
        widxf = jnp.min(jnp.where(scores == wm, gidxs, BIG_F))
        rrow = jnp.min(jnp.where((scores == wm) & (gidxs == widxf), li, 16))
        wrec = plsc.load_gather(all_rec_v, [zero16 + rrow * 16 + li])
        wrec_v[...] = wrec

        wx1 = plsc.load_gather(wrec_v, [zero16 + 2])
        wy1 = plsc.load_gather(wrec_v, [zero16 + 3])
        wx2 = plsc.load_gather(wrec_v, [zero16 + 4])
        wy2 = plsc.load_gather(wrec_v, [zero16 + 5])
        war = plsc.load_gather(wrec_v, [zero16 + 7])

        valid = wm > NEG_INF
        vf = jnp.where(valid, 1.0, 0.0)
        perm = jnp.where(li < 4, li + 2, jnp.where(li == 4, 0, jnp.where(li == 5, 6, 8)))
        og = plsc.load_gather(wrec_v, [perm])
        outrec = jnp.where(li <= 4, og * vf,
                 jnp.where(li == 5, jnp.where(valid, og, -1.0), 0.0))
        out_v[pl.ds(k * 16, 16)] = outrec

        widx = widxf.astype(jnp.int32)
        wg = widx // W
        wc = widx % W

        @pl.when(jnp.logical_and(wg >= base_row, wg < base_row + RPT))
        def _():
            wpos = (wg - base_row) * GP + wc
            plsc.store_scatter(s_v, [zero16 + wpos], neginf16, mask=li == 0)

        g0 = jnp.maximum(wg - WRAD, base_row)
        g1 = jnp.minimum(wg + WRAD + 1, base_row + RPT)
        cl = jnp.maximum(wc - WRAD, 0)
        t0 = cl // 16
        ta = t0 * 16
        tb = jnp.minimum(t0 + 1, ROW_VREGS - 1) * 16

        def row_step(r, carry2):
            r_local = r - base_row
            rb = r_local * GP
            for toff in (ta, tb):
                sl = pl.ds(rb + toff, 16)
                sv = s_v[sl]
                ix1 = jnp.maximum(wx1, x1_v[sl])
                iy1 = jnp.maximum(wy1, y1_v[sl])
                ix2 = jnp.minimum(wx2, x2_v[sl])
                iy2 = jnp.minimum(wy2, y2_v[sl])
                inter = jnp.maximum(ix2 - ix1, 0.0) * jnp.maximum(iy2 - iy1, 0.0)
                iou = inter / (war + area_v[sl] - inter + 1e-8)
                s_v[sl] = jnp.where(iou > iou_thr, NEG_INF, sv)
            rl16 = zero16 + r_local
            cidx = plsc.load_gather(ridx_v, [rl16])
            crm = jnp.max(plsc.load_gather(rmax_v, [rl16]))
            cpos = (zero16 + rb - r * W) + cidx.astype(jnp.int32)
            alive = jnp.max(plsc.load_gather(s_v, [cpos]))

            @pl.when(jnp.logical_and(alive == NEG_INF, crm > NEG_INF))
            def _():
                rm, ri = scan_row(r_local)
                rmax_v[...] = jnp.where(li == r_local, rm, rmax_v[...])
                ridx_v[...] = jnp.where(li == r_local, ri, ridx_v[...])

            return carry2

        lax.fori_loop(g0, g1, row_step, 0)
        return carry

    lax.fori_loop(0, MAX_OUT, step, 0, unroll=False)

    @pl.when(jnp.logical_and(cid == 0, sid == 0))
    def _():
        pltpu.sync_copy(out_v, out_hbm)


@jax.jit
def _decode_nms(cls_t, ctr_t, reg_t, centers, score_threshold, iou_threshold):
    def pitch(a):
        padw = [(0, 0)] * (a.ndim - 2) + [(0, GR - H), (0, GP - W)]
        return jnp.pad(a, padw)

    cls_p = pitch(cls_t[0].T.reshape(NUM_CLASSES, H, W))
    ctr_p = pitch(ctr_t[0].reshape(H, W))
    reg_p = pitch(reg_t[0].T.reshape(4, H, W))
    cen_p = pitch(centers.T.reshape(2, H, W))
    thr = jnp.asarray(score_threshold, jnp.float32).reshape(1, 1)

    grid2d = jax.ShapeDtypeStruct((GR, GP), jnp.float32)
    s0, x1, y1, x2, y2, area, idsf = pl.pallas_call(
        _prep_body,
        out_shape=[grid2d] * 7,
        in_specs=[
            pl.BlockSpec(memory_space=pltpu.VMEM),
            pl.BlockSpec(memory_space=pltpu.VMEM),
            pl.BlockSpec(memory_space=pltpu.VMEM),
            pl.BlockSpec(memory_space=pltpu.VMEM),
            pl.BlockSpec(memory_space=pltpu.SMEM),
        ],
        out_specs=[pl.BlockSpec(memory_space=pltpu.VMEM)] * 7,
    )(cls_p, ctr_p, reg_p, cen_p, thr)

    par = jnp.full((16,), jnp.asarray(iou_threshold, jnp.float32))

    nms = pl.kernel(
        _nms_sc_body,
        out_type=jax.ShapeDtypeStruct((OUT_ROWS * 16,), jnp.float32),
        mesh=plsc.VectorSubcoreMesh(core_axis_name="c", subcore_axis_name="s"),
        compiler_params=pltpu.CompilerParams(needs_layout_passes=False),
        scratch_types=[
            pltpu.VMEM((STRIPE,), jnp.float32),
            pltpu.VMEM((STRIPE,), jnp.float32),
            pltpu.VMEM((STRIPE,), jnp.float32),
            pltpu.VMEM((STRIPE,), jnp.float32),
            pltpu.VMEM((STRIPE,), jnp.float32),
            pltpu.VMEM((STRIPE,), jnp.float32),
            pltpu.VMEM((STRIPE,), jnp.float32),
            pltpu.VMEM((16,), jnp.float32),
            pltpu.VMEM((16,), jnp.float32),
            pltpu.VMEM((NT * 16,), jnp.float32),
            pltpu.VMEM((16,), jnp.float32),
            pltpu.VMEM((OUT_ROWS * 16,), jnp.float32),
            pltpu.VMEM((16,), jnp.float32),
            pltpu.VMEM((16,), jnp.float32),
            pltpu.VMEM_SHARED((2 * NT * 16,), jnp.float32),
        ],
    )

    out = nms(s0.reshape(NPIX), x1.reshape(NPIX), y1.reshape(NPIX),
              x2.reshape(NPIX), y2.reshape(NPIX), area.reshape(NPIX),
              idsf.reshape(NPIX), par)

    sel = out.reshape(OUT_ROWS, 16)[:MAX_OUT]
    out_boxes = sel[:, 0:4][None]
    out_scores = sel[:, 4][None]
    out_ids = sel[:, 5].astype(jnp.int32)[None]
    return out_boxes, out_scores, out_ids


def kernel(cls_target, ctr_target, reg_target, centers, score_threshold, iou_threshold):
    return _decode_nms(cls_target, ctr_target, reg_target, centers,
                       score_threshold, iou_threshold)


# speedup vs baseline: 1.1018x; 1.1018x over previous
import functools

import jax
import jax.numpy as jnp
from jax import lax
from jax.experimental import pallas as pl
from jax.experimental.pallas import tpu as pltpu
from jax.experimental.pallas import tpu_sc as plsc

H = 100
W = 200
N = H * W
NUM_CLASSES = 80
MAX_OUT = 300
NEG_INF = float("-inf")
BIG_F = 1.0e9

GR = 112
GP = 256
NPIX = GR * GP
NT = 16
RPT = GR // NT
STRIPE = RPT * GP
ROW_VREGS = 13
WRAD = 7
HALO_ROWS = RPT + 2 * WRAD
HSTRIPE = HALO_ROWS * GP
OUT_ROWS = 304


def _prep_body(cls_ref, ctr_ref, reg_ref, cen_ref, thr_ref,
               s_ref, x1_ref, y1_ref, x2_ref, y2_ref, area_ref, ids_ref):
    thr = thr_ref[0, 0]

    def cls_step(c, carry):
        acc, amax = carry
        x = cls_ref[c]
        gt = x > acc
        acc = jnp.where(gt, x, acc)
        amax = jnp.where(gt, c, amax)
        return acc, amax

    acc0 = cls_ref[0]
    amax0 = jnp.zeros((GR, GP), jnp.int32)
    cls_scores, cls_ids = jax.lax.fori_loop(1, NUM_CLASSES, cls_step, (acc0, amax0))

    score = cls_scores * ctr_ref[...]
    gi = jax.lax.broadcasted_iota(jnp.int32, (GR, GP), 0)
    ci = jax.lax.broadcasted_iota(jnp.int32, (GR, GP), 1)
    valid = (gi < H) & (ci < W)
    s_ref[...] = jnp.where((score > thr) & valid, score, NEG_INF)

    cx = cen_ref[0]
    cy = cen_ref[1]
    x1 = cx - reg_ref[0]
    y1 = cy - reg_ref[1]
    x2 = cx + reg_ref[2]
    y2 = cy + reg_ref[3]
    x1_ref[...] = x1
    y1_ref[...] = y1
    x2_ref[...] = x2
    y2_ref[...] = y2
    area_ref[...] = (x2 - x1) * (y2 - y1)
    ids_ref[...] = cls_ids.astype(jnp.float32)


def _nms_sc_body(s_hbm, x1_hbm, y1_hbm, x2_hbm, y2_hbm, area_hbm,
                 par_hbm, out_hbm,
                 s_v, x1_v, y1_v, x2_v, y2_v, area_v,
                 par_v, rec_v, all_rec_v, out_v,
                 rmax_v, ridx_v, shared):
    cid = lax.axis_index("c")
    sid = lax.axis_index("s")
    base = sid * STRIPE
    base_row = sid * RPT
    hstart = jnp.clip(base_row - WRAD, 0, GR - HALO_ROWS)
    dh = base_row - hstart

    pltpu.sync_copy(s_hbm.at[pl.ds(base, STRIPE)], s_v)
    hbase = hstart * GP
    pltpu.sync_copy(x1_hbm.at[pl.ds(hbase, HSTRIPE)], x1_v)
    pltpu.sync_copy(y1_hbm.at[pl.ds(hbase, HSTRIPE)], y1_v)
    pltpu.sync_copy(x2_hbm.at[pl.ds(hbase, HSTRIPE)], x2_v)
    pltpu.sync_copy(y2_hbm.at[pl.ds(hbase, HSTRIPE)], y2_v)
    pltpu.sync_copy(area_hbm.at[pl.ds(hbase, HSTRIPE)], area_v)
    pltpu.sync_copy(par_hbm, par_v)

    li = lax.iota(jnp.int32, 16)
    lif = li.astype(jnp.float32)
    iou_thr = par_v[...]
    zero16 = jnp.zeros((16,), jnp.int32)
    neginf16 = jnp.full((16,), NEG_INF, jnp.float32)

    def scan_row(r_local):
        g_orig = (base_row + r_local) * W
        rb = r_local * GP
        acc = s_v[pl.ds(rb, 16)]
        idxv = jnp.float32(1.0) * g_orig + lif
        for t in range(1, ROW_VREGS):
            sv = s_v[pl.ds(rb + t * 16, 16)]
            gt = sv > acc
            acc = jnp.where(gt, sv, acc)
            idxv = jnp.where(gt, jnp.float32(1.0) * (g_orig + t * 16) + lif, idxv)
        rm = jnp.max(acc)
        ri = jnp.min(jnp.where(acc == rm, idxv, BIG_F))
        return rm, ri

    rmax = neginf16
    ridx = jnp.full((16,), BIG_F, jnp.float32)
    for r in range(RPT):
        rm, ri = scan_row(r)
        rmax = jnp.where(li == r, rm, rmax)
        ridx = jnp.where(li == r, ri, ridx)
    rmax_v[...] = rmax
    ridx_v[...] = ridx

    def step(k, carry):
        rmax = rmax_v[...]
        ridx = ridx_v[...]
        m = jnp.max(rmax)
        idxf = jnp.min(jnp.where(rmax == m, ridx, BIG_F))
        rec_v[...] = jnp.where(li == 0, m, jnp.where(li == 1, idxf, 0.0))

        par = lax.rem(k, 2)
        slot = par * (NT * 8) + sid * 8
        pltpu.sync_copy(rec_v.at[pl.ds(0, 8)], shared.at[pl.ds(slot, 8)])
        plsc.subcore_barrier()
        pltpu.sync_copy(shared.at[pl.ds(par * (NT * 8), NT * 8)], all_rec_v)

        scores = plsc.load_gather(all_rec_v, [li * 8])
        gidxs = plsc.load_gather(all_rec_v, [li * 8 + 1])
        wm = jnp.max(scores)
        widxf = jnp.min(jnp.where(scores == wm, gidxs, BIG_F))

        valid = wm > NEG_INF
        vf = jnp.where(valid, 1.0, 0.0)
        outrec = jnp.where(li == 0, wm * vf,
                 jnp.where(li == 1, widxf, jnp.where(li == 2, vf, 0.0)))
        out_v[pl.ds(k * 16, 16)] = outrec

        widx = widxf.astype(jnp.int32)
        wg = widx // W
        wc = widx % W
        affected = jnp.logical_and(wg >= base_row - WRAD, wg < base_row + RPT + WRAD)

        @pl.when(affected)
        def _():
            @pl.when(jnp.logical_and(wg >= base_row, wg < base_row + RPT))
            def _():
                wpos = (wg - base_row) * GP + wc
                plsc.store_scatter(s_v, [zero16 + wpos], neginf16, mask=li == 0)

            hw = zero16 + ((wg - hstart) * GP + wc)
            wx1 = plsc.load_gather(x1_v, [hw])
            wy1 = plsc.load_gather(y1_v, [hw])
            wx2 = plsc.load_gather(x2_v, [hw])
            wy2 = plsc.load_gather(y2_v, [hw])
            war = plsc.load_gather(area_v, [hw])

            cl = jnp.maximum(wc - WRAD, 0)
            t0 = cl // 16
            ta = t0 * 16
            tb = jnp.minimum(t0 + 1, ROW_VREGS - 1) * 16

            for r in range(RPT):
                gr = base_row + r
                rw = jnp.logical_and(gr >= wg - WRAD, gr <= wg + WRAD)
                rb = r * GP
                hb = (r + dh) * GP
                for toff in (ta, tb):
                    sl = pl.ds(rb + toff, 16)
                    hl = pl.ds(hb + toff, 16)
                    sv = s_v[sl]
                    ix1 = jnp.maximum(wx1, x1_v[hl])
                    iy1 = jnp.maximum(wy1, y1_v[hl])
                    ix2 = jnp.minimum(wx2, x2_v[hl])
                    iy2 = jnp.minimum(wy2, y2_v[hl])
                    inter = jnp.maximum(ix2 - ix1, 0.0) * jnp.maximum(iy2 - iy1, 0.0)
                    iou = inter / (war + area_v[hl] - inter + 1e-8)
                    s_v[sl] = jnp.where((iou > iou_thr) & rw, NEG_INF, sv)

            ridx_now = ridx_v[...]
            rmax_now = rmax_v[...]
            rowbase = (zero16 + base_row + li) * W
            cpos = jnp.clip(li * GP + ridx_now.astype(jnp.int32) - rowbase,
                            0, STRIPE - 1)
            avals = plsc.load_gather(s_v, [cpos])
            dead = ((avals == NEG_INF) & (rmax_now > NEG_INF) & (li < RPT)
                    ).astype(jnp.int32)

            for r in range(RPT):
                @pl.when(dead[r] == 1)
                def _(r=r):
                    rm, ri = scan_row(r)
                    rmax_v[...] = jnp.where(li == r, rm, rmax_v[...])
                    ridx_v[...] = jnp.where(li == r, ri, ridx_v[...])

        return carry

    lax.fori_loop(0, MAX_OUT, step, 0, unroll=False)

    @pl.when(jnp.logical_and(cid == 0, sid == 0))
    def _():
        pltpu.sync_copy(out_v, out_hbm)


@jax.jit
def _decode_nms(cls_t, ctr_t, reg_t, centers, score_threshold, iou_threshold):
    def pitch(a):
        padw = [(0, 0)] * (a.ndim - 2) + [(0, GR - H), (0, GP - W)]
        return jnp.pad(a, padw)

    cls_p = pitch(cls_t[0].T.reshape(NUM_CLASSES, H, W))
    ctr_p = pitch(ctr_t[0].reshape(H, W))
    reg_p = pitch(reg_t[0].T.reshape(4, H, W))
    cen_p = pitch(centers.T.reshape(2, H, W))
    thr = jnp.asarray(score_threshold, jnp.float32).reshape(1, 1)

    grid2d = jax.ShapeDtypeStruct((GR, GP), jnp.float32)
    s0, x1, y1, x2, y2, area, idsf = pl.pallas_call(
        _prep_body,
        out_shape=[grid2d] * 7,
        in_specs=[
            pl.BlockSpec(memory_space=pltpu.VMEM),
            pl.BlockSpec(memory_space=pltpu.VMEM),
            pl.BlockSpec(memory_space=pltpu.VMEM),
            pl.BlockSpec(memory_space=pltpu.VMEM),
            pl.BlockSpec(memory_space=pltpu.SMEM),
        ],
        out_specs=[pl.BlockSpec(memory_space=pltpu.VMEM)] * 7,
    )(cls_p, ctr_p, reg_p, cen_p, thr)

    par = jnp.full((16,), jnp.asarray(iou_threshold, jnp.float32))

    nms = pl.kernel(
        _nms_sc_body,
        out_type=jax.ShapeDtypeStruct((OUT_ROWS * 16,), jnp.float32),
        mesh=plsc.VectorSubcoreMesh(core_axis_name="c", subcore_axis_name="s"),
        compiler_params=pltpu.CompilerParams(needs_layout_passes=False),
        scratch_types=[
            pltpu.VMEM((STRIPE,), jnp.float32),
            pltpu.VMEM((HSTRIPE,), jnp.float32),
            pltpu.VMEM((HSTRIPE,), jnp.float32),
            pltpu.VMEM((HSTRIPE,), jnp.float32),
            pltpu.VMEM((HSTRIPE,), jnp.float32),
            pltpu.VMEM((HSTRIPE,), jnp.float32),
            pltpu.VMEM((16,), jnp.float32),
            pltpu.VMEM((16,), jnp.float32),
            pltpu.VMEM((NT * 8,), jnp.float32),
            pltpu.VMEM((OUT_ROWS * 16,), jnp.float32),
            pltpu.VMEM((16,), jnp.float32),
            pltpu.VMEM((16,), jnp.float32),
            pltpu.VMEM_SHARED((2 * NT * 8,), jnp.float32),
        ],
    )

    out = nms(s0.reshape(NPIX), x1.reshape(NPIX), y1.reshape(NPIX),
              x2.reshape(NPIX), y2.reshape(NPIX), area.reshape(NPIX), par)

    sel = out.reshape(OUT_ROWS, 16)[:MAX_OUT]
    out_scores = sel[:, 0][None]
    widx = sel[:, 1].astype(jnp.int32)
    vmask = sel[:, 2] > 0.0
    pidx = (widx // W) * GP + widx % W
    bx = jnp.stack([x1.reshape(NPIX)[pidx], y1.reshape(NPIX)[pidx],
                    x2.reshape(NPIX)[pidx], y2.reshape(NPIX)[pidx]], axis=-1)
    out_boxes = jnp.where(vmask[:, None], bx, 0.0)[None]
    out_ids = jnp.where(vmask, idsf.reshape(NPIX)[pidx].astype(jnp.int32), -1)[None]
    return out_boxes, out_scores, out_ids


def kernel(cls_target, ctr_target, reg_target, centers, score_threshold, iou_threshold):
    return _decode_nms(cls_target, ctr_target, reg_target, centers,
                       score_threshold, iou_threshold)
